# baseline (speedup 1.0000x reference)
import functools

import jax
import jax.numpy as jnp
from jax import lax
from jax.experimental import pallas as pl
from jax.experimental.pallas import tpu as pltpu
from jax.experimental.pallas import tpu_sc as plsc

NC = 2
NS = 16
NW = NC * NS

B = 4096
C = 200
D = 32
BPW = B // NW
CP = 208
G1 = 104
G2 = C - G1


def _fpmc_body(user_h, last_h, cand_h, uemb_h, lemb_h, nemb_h, out_h,
               uidx_v, lidx_v, cidx_v, ubuf, lbuf, gbuf, outbuf,
               sem_g, sem_o):
    wid = lax.axis_index("s") * NC + lax.axis_index("c")
    base = pl.multiple_of(wid * BPW, 8)
    cbase = pl.multiple_of(wid * (BPW * C), 8)

    pltpu.sync_copy(user_h.at[pl.ds(base, BPW)], uidx_v)
    pltpu.sync_copy(last_h.at[pl.ds(base, BPW)], lidx_v)
    pltpu.sync_copy(cand_h.at[pl.ds(cbase, BPW * C)], cidx_v)
    cu = pltpu.async_copy(uemb_h.at[uidx_v], ubuf, sem_g)
    cl = pltpu.async_copy(lemb_h.at[lidx_v], lbuf, sem_g)
    cu.wait()
    cl.wait()

    lane = lax.iota(jnp.int32, 16)
    lane16 = lane + 16
    col_idx = [jnp.full((16,), d, jnp.int32) for d in range(D)]

    def body(b, carry):
        offc = pl.multiple_of(b * C, 8)
        c1 = pltpu.async_copy(nemb_h.at[cidx_v.at[pl.ds(offc, G1)]],
                              gbuf.at[pl.ds(0, G1)], sem_g)
        c2 = pltpu.async_copy(
            nemb_h.at[cidx_v.at[pl.ds(pl.multiple_of(offc + G1, 8), G2)]],
            gbuf.at[pl.ds(G1, G2)], sem_g)

        bb = jnp.full((16,), b, jnp.int32)
        u0 = plsc.load_gather(ubuf, [bb, lane])
        u1 = plsc.load_gather(ubuf, [bb, lane16])
        l0 = plsc.load_gather(lbuf, [bb, lane])
        l1 = plsc.load_gather(lbuf, [bb, lane16])
        s = jnp.sum(u0 * l0 + u1 * l1)
        w0 = u0 + l0
        w1 = u1 + l1
        ws = [w0[d] for d in range(16)] + [w1[d] for d in range(16)]

        c1.wait()
        c2.wait()

        obase = pl.multiple_of(b * CP, 8)
        for grp in range(CP // 16):
            row_idx = lane + (grp * 16)
            acc = jnp.full((16,), s, jnp.float32)
            for d in range(D):
                g = plsc.load_gather(gbuf, [row_idx, col_idx[d]])
                acc = acc + ws[d] * g
            outbuf[pl.ds(pl.multiple_of(obase + grp * 16, 8), 16)] = acc

        pltpu.async_copy(
            outbuf.at[pl.ds(obase, C)],
            out_h.at[pl.ds(pl.multiple_of(cbase + b * C, 8), C)], sem_o)
        return carry

    lax.fori_loop(0, BPW, body, 0)

    def drain(b, carry):
        obase = pl.multiple_of(b * CP, 8)
        pltpu.make_async_copy(
            outbuf.at[pl.ds(obase, C)],
            out_h.at[pl.ds(pl.multiple_of(cbase + b * C, 8), C)],
            sem_o).wait()
        return carry

    lax.fori_loop(0, BPW, drain, 0)


@functools.partial(
    pl.kernel,
    out_type=jax.ShapeDtypeStruct((B * C,), jnp.float32),
    mesh=plsc.VectorSubcoreMesh(core_axis_name="c", subcore_axis_name="s"),
    compiler_params=pltpu.CompilerParams(
        needs_layout_passes=False, use_tc_tiling_on_sc=False),
    scratch_types=[
        pltpu.VMEM((BPW,), jnp.int32),
        pltpu.VMEM((BPW,), jnp.int32),
        pltpu.VMEM((BPW * C,), jnp.int32),
        pltpu.VMEM((BPW, D), jnp.float32),
        pltpu.VMEM((BPW, D), jnp.float32),
        pltpu.VMEM((CP, D), jnp.float32),
        pltpu.VMEM((BPW * CP,), jnp.float32),
        pltpu.SemaphoreType.DMA,
        pltpu.SemaphoreType.DMA,
    ],
)
def _fpmc_sc(user_h, last_h, cand_h, uemb_h, lemb_h, nemb_h, out_h,
             uidx_v, lidx_v, cidx_v, ubuf, lbuf, gbuf, outbuf,
             sem_g, sem_o):
    _fpmc_body(user_h, last_h, cand_h, uemb_h, lemb_h, nemb_h, out_h,
               uidx_v, lidx_v, cidx_v, ubuf, lbuf, gbuf, outbuf,
               sem_g, sem_o)


@jax.jit
def kernel(user, last, candidates, user_emb, last_item_emb, next_item_emb):
    user = user.reshape(B).astype(jnp.int32)
    last = last.reshape(B).astype(jnp.int32)
    candidates = candidates.reshape(B * C).astype(jnp.int32)
    out = _fpmc_sc(user, last, candidates, user_emb, last_item_emb,
                   next_item_emb)
    return out.reshape(B, C)


# speedup vs baseline: 1.0381x; 1.0381x over previous
import functools

import jax
import jax.numpy as jnp
from jax import lax
from jax.experimental import pallas as pl
from jax.experimental.pallas import tpu as pltpu
from jax.experimental.pallas import tpu_sc as plsc

NC = 2
NS = 16
NW = NC * NS

B = 4096
C = 200
D = 32
BPW = B // NW
NG = 13
CP = NG * 16


def _fpmc_body(user_h, last_h, cand_h, uemb_h, lemb_h, nemb_h, out_h,
               uidx_v, lidx_v, cidx_v, ubuf, lbuf, gbuf0, gbuf1, outbuf,
               sem_g0, sem_g1, sem_u):
    wid = lax.axis_index("s") * NC + lax.axis_index("c")
    base = pl.multiple_of(wid * BPW, 8)
    cbase = pl.multiple_of(wid * (BPW * C), 8)

    pltpu.sync_copy(user_h.at[pl.ds(base, BPW)], uidx_v)
    pltpu.sync_copy(last_h.at[pl.ds(base, BPW)], lidx_v)
    pltpu.sync_copy(cand_h.at[pl.ds(cbase, BPW * C)], cidx_v)
    cu = pltpu.async_copy(uemb_h.at[uidx_v], ubuf, sem_u)
    cl = pltpu.async_copy(lemb_h.at[lidx_v], lbuf, sem_u)
    cu.wait()
    cl.wait()

    lane = lax.iota(jnp.int32, 16)
    lane16 = lane + 16
    tail_mask = lane < (C - (NG - 1) * 16)
    col_idx = [jnp.full((16,), d, jnp.int32) for d in range(D)]

    def fire(b, gbuf, sem):
        offc = pl.multiple_of(b * C, 8)
        pltpu.async_copy(nemb_h.at[cidx_v.at[pl.ds(offc, C)]],
                         gbuf.at[pl.ds(0, C)], sem)

    def wait_fire(gbuf, sem):
        pltpu.make_async_copy(nemb_h.at[cidx_v.at[pl.ds(0, C)]],
                              gbuf.at[pl.ds(0, C)], sem).wait()

    def compute(b, gbuf):
        bb = jnp.full((16,), b, jnp.int32)
        u0 = plsc.load_gather(ubuf, [bb, lane])
        u1 = plsc.load_gather(ubuf, [bb, lane16])
        l0 = plsc.load_gather(lbuf, [bb, lane])
        l1 = plsc.load_gather(lbuf, [bb, lane16])
        s = jnp.sum(u0 * l0 + u1 * l1)
        w0 = u0 + l0
        w1 = u1 + l1
        ws = [w0[d] for d in range(16)] + [w1[d] for d in range(16)]

        ob = pl.multiple_of(b * C, 8)
        for grp in range(NG):
            row_idx = lane + (grp * 16)
            acc = jnp.full((16,), s, jnp.float32)
            for d in range(D):
                g = plsc.load_gather(gbuf, [row_idx, col_idx[d]])
                acc = acc + ws[d] * g
            if grp < NG - 1:
                outbuf[pl.ds(pl.multiple_of(ob + grp * 16, 8), 16)] = acc
            else:
                plsc.store_compressed(
                    outbuf.at[pl.ds(pl.multiple_of(ob + grp * 16, 8), 16)],
                    acc, mask=tail_mask)

    fire(0, gbuf0, sem_g0)

    def body(g, carry):
        b0 = g * 2
        b1 = b0 + 1
        fire(b1, gbuf1, sem_g1)
        wait_fire(gbuf0, sem_g0)
        compute(b0, gbuf0)
        fire(jnp.minimum(b0 + 2, BPW - 1), gbuf0, sem_g0)
        wait_fire(gbuf1, sem_g1)
        compute(b1, gbuf1)
        return carry

    lax.fori_loop(0, BPW // 2, body, 0)
    wait_fire(gbuf0, sem_g0)

    pltpu.sync_copy(outbuf.at[pl.ds(0, BPW * C)],
                    out_h.at[pl.ds(cbase, BPW * C)])


@functools.partial(
    pl.kernel,
    out_type=jax.ShapeDtypeStruct((B * C,), jnp.float32),
    mesh=plsc.VectorSubcoreMesh(core_axis_name="c", subcore_axis_name="s"),
    compiler_params=pltpu.CompilerParams(
        needs_layout_passes=False, use_tc_tiling_on_sc=False),
    scratch_types=[
        pltpu.VMEM((BPW,), jnp.int32),
        pltpu.VMEM((BPW,), jnp.int32),
        pltpu.VMEM((BPW * C,), jnp.int32),
        pltpu.VMEM((BPW, D), jnp.float32),
        pltpu.VMEM((BPW, D), jnp.float32),
        pltpu.VMEM((CP, D), jnp.float32),
        pltpu.VMEM((CP, D), jnp.float32),
        pltpu.VMEM((BPW * C + 8,), jnp.float32),
        pltpu.SemaphoreType.DMA,
        pltpu.SemaphoreType.DMA,
        pltpu.SemaphoreType.DMA,
    ],
)
def _fpmc_sc(user_h, last_h, cand_h, uemb_h, lemb_h, nemb_h, out_h,
             uidx_v, lidx_v, cidx_v, ubuf, lbuf, gbuf0, gbuf1, outbuf,
             sem_g0, sem_g1, sem_u):
    _fpmc_body(user_h, last_h, cand_h, uemb_h, lemb_h, nemb_h, out_h,
               uidx_v, lidx_v, cidx_v, ubuf, lbuf, gbuf0, gbuf1, outbuf,
               sem_g0, sem_g1, sem_u)


@jax.jit
def kernel(user, last, candidates, user_emb, last_item_emb, next_item_emb):
    user = user.reshape(B).astype(jnp.int32)
    last = last.reshape(B).astype(jnp.int32)
    candidates = candidates.reshape(B * C).astype(jnp.int32)
    out = _fpmc_sc(user, last, candidates, user_emb, last_item_emb,
                   next_item_emb)
    return out.reshape(B, C)


# speedup vs baseline: 1.8250x; 1.7580x over previous
import functools

import jax
import jax.numpy as jnp
from jax import lax
from jax.experimental import pallas as pl
from jax.experimental.pallas import tpu as pltpu
from jax.experimental.pallas import tpu_sc as plsc

NC = 2
NS = 16
NW = NC * NS

B = 4096
C = 200
D = 32
BPW = B // NW
NG = 13
CP = NG * 16
G1 = 104
G2 = C - G1


def _fpmc_body(urows_h, lrows_h, cand_h, nemb_h, out_h,
               cidx_v, ubuf, lbuf, gbuf0, gbuf1, outbuf,
               sem_g0, sem_g1, sem_u):
    wid = lax.axis_index("s") * NC + lax.axis_index("c")
    dbase = pl.multiple_of(wid * (BPW * D), 8)
    cbase = pl.multiple_of(wid * (BPW * C), 8)

    pltpu.sync_copy(cand_h.at[pl.ds(cbase, BPW * C)], cidx_v)
    cu = pltpu.async_copy(urows_h.at[pl.ds(dbase, BPW * D)], ubuf, sem_u)
    cl = pltpu.async_copy(lrows_h.at[pl.ds(dbase, BPW * D)], lbuf, sem_u)
    cu.wait()
    cl.wait()

    lane = lax.iota(jnp.int32, 16)
    tail_mask = lane < (C - (NG - 1) * 16)
    col_idx = [jnp.full((16,), d, jnp.int32) for d in range(D)]

    def fire(b, gbuf, sem):
        offc = pl.multiple_of(b * C, 8)
        pltpu.async_copy(nemb_h.at[cidx_v.at[pl.ds(offc, G1)]],
                         gbuf.at[pl.ds(0, G1)], sem)
        pltpu.async_copy(
            nemb_h.at[cidx_v.at[pl.ds(pl.multiple_of(offc + G1, 8), G2)]],
            gbuf.at[pl.ds(G1, G2)], sem)

    def wait_fire(gbuf, sem):
        pltpu.make_async_copy(nemb_h.at[cidx_v.at[pl.ds(0, G1)]],
                              gbuf.at[pl.ds(0, G1)], sem).wait()
        pltpu.make_async_copy(nemb_h.at[cidx_v.at[pl.ds(0, G2)]],
                              gbuf.at[pl.ds(G1, G2)], sem).wait()

    def compute(b, gbuf):
        bd = pl.multiple_of(b * D, 8)
        bb = jnp.full((16,), bd, jnp.int32) + lane
        u0 = plsc.load_gather(ubuf, [bb])
        u1 = plsc.load_gather(ubuf, [bb + 16])
        l0 = plsc.load_gather(lbuf, [bb])
        l1 = plsc.load_gather(lbuf, [bb + 16])
        s = jnp.sum(u0 * l0 + u1 * l1)
        w0 = u0 + l0
        w1 = u1 + l1
        ws = [w0[d] for d in range(16)] + [w1[d] for d in range(16)]

        ob = pl.multiple_of(b * C, 8)
        for grp in range(NG):
            row_idx = lane + (grp * 16)
            acc = jnp.full((16,), s, jnp.float32)
            for d in range(D):
                g = plsc.load_gather(gbuf, [row_idx, col_idx[d]])
                acc = acc + ws[d] * g
            if grp < NG - 1:
                outbuf[pl.ds(pl.multiple_of(ob + grp * 16, 8), 16)] = acc
            else:
                plsc.store_compressed(
                    outbuf.at[pl.ds(pl.multiple_of(ob + grp * 16, 8), 16)],
                    acc, mask=tail_mask)

    fire(0, gbuf0, sem_g0)

    def body(g, carry):
        b0 = g * 2
        b1 = b0 + 1
        fire(b1, gbuf1, sem_g1)
        wait_fire(gbuf0, sem_g0)
        compute(b0, gbuf0)
        fire(jnp.minimum(b0 + 2, BPW - 1), gbuf0, sem_g0)
        wait_fire(gbuf1, sem_g1)
        compute(b1, gbuf1)
        return carry

    lax.fori_loop(0, BPW // 2, body, 0)
    wait_fire(gbuf0, sem_g0)

    pltpu.sync_copy(outbuf.at[pl.ds(0, BPW * C)],
                    out_h.at[pl.ds(cbase, BPW * C)])


@functools.partial(
    pl.kernel,
    out_type=jax.ShapeDtypeStruct((B * C,), jnp.float32),
    mesh=plsc.VectorSubcoreMesh(core_axis_name="c", subcore_axis_name="s"),
    compiler_params=pltpu.CompilerParams(
        needs_layout_passes=False, use_tc_tiling_on_sc=False),
    scratch_types=[
        pltpu.VMEM((BPW * C,), jnp.int32),
        pltpu.VMEM((BPW * D,), jnp.float32),
        pltpu.VMEM((BPW * D,), jnp.float32),
        pltpu.VMEM((CP, D), jnp.float32),
        pltpu.VMEM((CP, D), jnp.float32),
        pltpu.VMEM((BPW * C + 8,), jnp.float32),
        pltpu.SemaphoreType.DMA,
        pltpu.SemaphoreType.DMA,
        pltpu.SemaphoreType.DMA,
    ],
)
def _fpmc_sc(urows_h, lrows_h, cand_h, nemb_h, out_h,
             cidx_v, ubuf, lbuf, gbuf0, gbuf1, outbuf,
             sem_g0, sem_g1, sem_u):
    _fpmc_body(urows_h, lrows_h, cand_h, nemb_h, out_h,
               cidx_v, ubuf, lbuf, gbuf0, gbuf1, outbuf,
               sem_g0, sem_g1, sem_u)


@jax.jit
def kernel(user, last, candidates, user_emb, last_item_emb, next_item_emb):
    user = user.reshape(B).astype(jnp.int32)
    last = last.reshape(B).astype(jnp.int32)
    candidates = candidates.reshape(B * C).astype(jnp.int32)
    u_rows = jnp.take(user_emb, user, axis=0).reshape(B * D)
    l_rows = jnp.take(last_item_emb, last, axis=0).reshape(B * D)
    out = _fpmc_sc(u_rows, l_rows, candidates, next_item_emb)
    return out.reshape(B, C)


# speedup vs baseline: 1.8402x; 1.0083x over previous
import functools

import jax
import jax.numpy as jnp
from jax import lax
from jax.experimental import pallas as pl
from jax.experimental.pallas import tpu as pltpu
from jax.experimental.pallas import tpu_sc as plsc

NC = 2
NS = 16
NW = NC * NS

B = 4096
C = 200
D = 32
BPW = B // NW
NG = 13
CP = NG * 16
G1 = 104
G2 = C - G1


def _fpmc_body(urows_h, lrows_h, cand_h, nemb_h, out_h,
               cidx_v, ubuf, lbuf, gbuf0, gbuf1, outbuf,
               sem_g0, sem_g1, sem_u):
    wid = lax.axis_index("s") * NC + lax.axis_index("c")
    dbase = pl.multiple_of(wid * (BPW * D), 8)
    cbase = pl.multiple_of(wid * (BPW * C), 8)

    pltpu.sync_copy(cand_h.at[pl.ds(cbase, BPW * C)], cidx_v)
    cu = pltpu.async_copy(urows_h.at[pl.ds(dbase, BPW * D)], ubuf, sem_u)
    cl = pltpu.async_copy(lrows_h.at[pl.ds(dbase, BPW * D)], lbuf, sem_u)
    cu.wait()
    cl.wait()

    lane = lax.iota(jnp.int32, 16)
    tail_mask = lane < (C - (NG - 1) * 16)
    col_idx = [jnp.full((16,), d, jnp.int32) for d in range(D)]

    def fire(b, gbuf, sem):
        offc = pl.multiple_of(b * C, 8)
        pltpu.async_copy(nemb_h.at[cidx_v.at[pl.ds(offc, G1)]],
                         gbuf.at[pl.ds(0, G1)], sem)
        pltpu.async_copy(
            nemb_h.at[cidx_v.at[pl.ds(pl.multiple_of(offc + G1, 8), G2)]],
            gbuf.at[pl.ds(G1, G2)], sem)

    def wait_fire(gbuf, sem):
        pltpu.make_async_copy(nemb_h.at[cidx_v.at[pl.ds(0, G1)]],
                              gbuf.at[pl.ds(0, G1)], sem).wait()
        pltpu.make_async_copy(nemb_h.at[cidx_v.at[pl.ds(0, G2)]],
                              gbuf.at[pl.ds(G1, G2)], sem).wait()

    def compute(b, gbuf):
        bd = pl.multiple_of(b * D, 8)
        bb = jnp.full((16,), bd, jnp.int32) + lane
        u0 = plsc.load_gather(ubuf, [bb])
        u1 = plsc.load_gather(ubuf, [bb + 16])
        l0 = plsc.load_gather(lbuf, [bb])
        l1 = plsc.load_gather(lbuf, [bb + 16])
        s = jnp.sum(u0 * l0 + u1 * l1)
        w0 = u0 + l0
        w1 = u1 + l1
        sv = jnp.full((16,), s, jnp.float32)
        ws = [w0.at[col_idx[d]].get(mode="promise_in_bounds")
              for d in range(16)]
        ws += [w1.at[col_idx[d]].get(mode="promise_in_bounds")
               for d in range(16)]

        ob = pl.multiple_of(b * C, 8)
        for grp in range(NG):
            row_idx = lane + (grp * 16)
            acc = sv
            for d in range(D):
                g = plsc.load_gather(gbuf, [row_idx, col_idx[d]])
                acc = acc + ws[d] * g
            if grp < NG - 1:
                outbuf[pl.ds(pl.multiple_of(ob + grp * 16, 8), 16)] = acc
            else:
                plsc.store_compressed(
                    outbuf.at[pl.ds(pl.multiple_of(ob + grp * 16, 8), 16)],
                    acc, mask=tail_mask)

    fire(0, gbuf0, sem_g0)

    def body(g, carry):
        b0 = g * 2
        b1 = b0 + 1
        fire(b1, gbuf1, sem_g1)
        wait_fire(gbuf0, sem_g0)
        compute(b0, gbuf0)
        fire(jnp.minimum(b0 + 2, BPW - 1), gbuf0, sem_g0)
        wait_fire(gbuf1, sem_g1)
        compute(b1, gbuf1)
        return carry

    lax.fori_loop(0, BPW // 2, body, 0)
    wait_fire(gbuf0, sem_g0)

    pltpu.sync_copy(outbuf.at[pl.ds(0, BPW * C)],
                    out_h.at[pl.ds(cbase, BPW * C)])


@functools.partial(
    pl.kernel,
    out_type=jax.ShapeDtypeStruct((B * C,), jnp.float32),
    mesh=plsc.VectorSubcoreMesh(core_axis_name="c", subcore_axis_name="s"),
    compiler_params=pltpu.CompilerParams(
        needs_layout_passes=False, use_tc_tiling_on_sc=False),
    scratch_types=[
        pltpu.VMEM((BPW * C,), jnp.int32),
        pltpu.VMEM((BPW * D,), jnp.float32),
        pltpu.VMEM((BPW * D,), jnp.float32),
        pltpu.VMEM((CP, D), jnp.float32),
        pltpu.VMEM((CP, D), jnp.float32),
        pltpu.VMEM((BPW * C + 8,), jnp.float32),
        pltpu.SemaphoreType.DMA,
        pltpu.SemaphoreType.DMA,
        pltpu.SemaphoreType.DMA,
    ],
)
def _fpmc_sc(urows_h, lrows_h, cand_h, nemb_h, out_h,
             cidx_v, ubuf, lbuf, gbuf0, gbuf1, outbuf,
             sem_g0, sem_g1, sem_u):
    _fpmc_body(urows_h, lrows_h, cand_h, nemb_h, out_h,
               cidx_v, ubuf, lbuf, gbuf0, gbuf1, outbuf,
               sem_g0, sem_g1, sem_u)


@jax.jit
def kernel(user, last, candidates, user_emb, last_item_emb, next_item_emb):
    user = user.reshape(B).astype(jnp.int32)
    last = last.reshape(B).astype(jnp.int32)
    candidates = candidates.reshape(B * C).astype(jnp.int32)
    u_rows = jnp.take(user_emb, user, axis=0).reshape(B * D)
    l_rows = jnp.take(last_item_emb, last, axis=0).reshape(B * D)
    out = _fpmc_sc(u_rows, l_rows, candidates, next_item_emb)
    return out.reshape(B, C)
